# Initial kernel scaffold; baseline (speedup 1.0000x reference)
import jax
import jax.numpy as jnp
from jax.experimental import pallas as pl


def kernel(position_ids, pos_emb):
    raise NotImplementedError("write your pallas kernel here")


# speedup vs baseline: 1.5176x; 1.5176x over previous
import functools

import jax
import jax.numpy as jnp
from jax import lax
from jax.experimental import pallas as pl
from jax.experimental.pallas import tpu as pltpu
from jax.experimental.pallas import tpu_sc as plsc

_TABLE_ROWS = 8192
_D = 1024
_B_TOTAL = 8192
_NC = 2
_NS = 16
_NW = _NC * _NS
_B_PER_W = _B_TOTAL // _NW
_CHUNK = 64
_N_CHUNKS = _B_PER_W // _CHUNK


def _gather_sc(table, idx):
  mesh = plsc.VectorSubcoreMesh(core_axis_name="c", subcore_axis_name="s")

  @functools.partial(
      pl.kernel,
      mesh=mesh,
      out_type=jax.ShapeDtypeStruct((_NW, _N_CHUNKS, _CHUNK, _D),
                                    jnp.float32),
      scratch_types=[
          pltpu.VMEM((_N_CHUNKS, _CHUNK), jnp.int32),
          pltpu.VMEM((_CHUNK, _D), jnp.float32),
          pltpu.SemaphoreType.DMA,
      ],
  )
  def k(table_hbm, idx_hbm, out_hbm, idx_v, buf, sem):
    wid = lax.axis_index("s") * _NC + lax.axis_index("c")
    pltpu.sync_copy(idx_hbm.at[wid], idx_v)
    for c in range(_N_CHUNKS):
      pltpu.async_copy(table_hbm.at[idx_v.at[c]], buf, sem).wait()
      pltpu.sync_copy(buf, out_hbm.at[wid, c])

  return k(table, idx)


def kernel(position_ids, pos_emb):
  batch, seq = position_ids.shape
  table = pos_emb.reshape(_TABLE_ROWS, _D)
  idx = position_ids.reshape(_NW, _N_CHUNKS, _CHUNK).astype(jnp.int32)
  out = _gather_sc(table, idx)
  return out.reshape(1, batch, seq, _D)
